# baseline (speedup 1.0000x reference)
import jax
import jax.numpy as jnp
from jax import lax
from jax.experimental import pallas as pl
from jax.experimental.pallas import tpu as pltpu
from jax.experimental.pallas import tpu_sc as plsc

VOCAB = 400000
EMBED_DIM = 50
DP = 64
BATCH = 4096
SEQ = 200

_INFO = plsc.get_sparse_core_info()
_NC = _INFO.num_cores
_NS = _INFO.num_subcores
_NW = _NC * _NS

_B = BATCH * SEQ
_PER_W = _B // _NW
_G = 128
_SUB = 4
_CHUNK = _G * _SUB
_STEPS = _PER_W // _CHUNK
_IDXR = _PER_W // _G


def _pad_body(t_ref, o_ref):
    o_ref[...] = jnp.pad(t_ref[...], ((0, 0), (0, DP - EMBED_DIM)))


def _unpad_body(t_ref, o_ref):
    o_ref[...] = t_ref[...][:, :EMBED_DIM]


def _gather_body(x_hbm, tab_hbm, out_hbm, idx_v, rows0, rows1, sem0, sem1):
    wid = lax.axis_index("s") * _NC + lax.axis_index("c")
    base = wid * _PER_W

    def fire(i, buf, sem):
        for j in range(_SUB):
            pltpu.async_copy(tab_hbm.at[idx_v.at[i * _SUB + j]],
                             buf.at[pl.ds(j * _G, _G)], sem)

    def drain(buf, sem):
        pltpu.make_async_copy(tab_hbm.at[pl.ds(0, _CHUNK)], buf, sem).wait()

    pltpu.sync_copy(x_hbm.at[pl.ds(wid * _IDXR, _IDXR)], idx_v)
    fire(0, rows0, sem0)

    def step(it, c):
        i0 = 2 * it
        fire(i0 + 1, rows1, sem1)
        drain(rows0, sem0)
        pltpu.sync_copy(rows0, out_hbm.at[pl.ds(base + i0 * _CHUNK, _CHUNK)])

        @pl.when(it < (_STEPS // 2 - 1))
        def _():
            fire(i0 + 2, rows0, sem0)

        drain(rows1, sem1)
        pltpu.sync_copy(rows1,
                        out_hbm.at[pl.ds(base + (i0 + 1) * _CHUNK, _CHUNK)])
        return c

    lax.fori_loop(0, _STEPS // 2, step, 0)


def kernel(x, emb_table):
    idx = x.reshape(_B // _G, _G).astype(jnp.int32)

    pad = pl.pallas_call(
        _pad_body,
        grid=(VOCAB // 2000,),
        in_specs=[pl.BlockSpec((2000, EMBED_DIM), lambda i: (i, 0))],
        out_specs=pl.BlockSpec((2000, DP), lambda i: (i, 0)),
        out_shape=jax.ShapeDtypeStruct((VOCAB, DP), jnp.float32),
    )
    tab64 = pad(emb_table)

    gather = pl.kernel(
        _gather_body,
        out_type=jax.ShapeDtypeStruct((_B, DP), jnp.float32),
        mesh=plsc.VectorSubcoreMesh(core_axis_name="c", subcore_axis_name="s"),
        scratch_types=[
            pltpu.VMEM((_IDXR, _G), jnp.int32),
            pltpu.VMEM((_CHUNK, DP), jnp.float32),
            pltpu.VMEM((_CHUNK, DP), jnp.float32),
            pltpu.SemaphoreType.DMA,
            pltpu.SemaphoreType.DMA,
        ],
        compiler_params=pltpu.CompilerParams(use_tc_tiling_on_sc=False),
    )
    out64 = gather(idx, tab64)

    unpad = pl.pallas_call(
        _unpad_body,
        grid=(_B // 4096,),
        in_specs=[pl.BlockSpec((4096, DP), lambda i: (i, 0))],
        out_specs=pl.BlockSpec((4096, EMBED_DIM), lambda i: (i, 0)),
        out_shape=jax.ShapeDtypeStruct((_B, EMBED_DIM), jnp.float32),
    )
    out = unpad(out64)
    return out.reshape(BATCH, SEQ, EMBED_DIM)
